# Initial kernel scaffold; baseline (speedup 1.0000x reference)
import jax
import jax.numpy as jnp
from jax.experimental import pallas as pl


def kernel(x, tables):
    raise NotImplementedError("write your pallas kernel here")


# speedup vs baseline: 1.2006x; 1.2006x over previous
import functools

import jax
import jax.numpy as jnp
from jax import lax
from jax.experimental import pallas as pl
from jax.experimental.pallas import tpu as pltpu
from jax.experimental.pallas import tpu_sc as plsc

_B = 16384
_F = 26
_V = 100000
_D = 32
_N = _B * _F
_NC = 2
_NS = 16
_NW = _NC * _NS
_PW = _N // _NW
_BLK = 128
_CHUNK_BLKS = 8
_CROWS = _CHUNK_BLKS * _BLK
_NCHUNK = _PW // _CROWS
_LANES = 16


def _gather_body(x_hbm, tab_hbm, out_hbm, idx_v, rows_v, sem):
    wid = lax.axis_index("s") * _NC + lax.axis_index("c")
    blk0 = wid * (_PW // _BLK)

    def chunk(c, carry):
        gb = blk0 + c * _CHUNK_BLKS
        pltpu.sync_copy(x_hbm.at[pl.ds(gb, _CHUNK_BLKS)], idx_v)
        lane = lax.iota(jnp.int32, _LANES)
        for j in range(_CHUNK_BLKS):
            row_base = (gb + j) * _BLK
            for v in range(_BLK // _LANES):
                pos = lane + (row_base + v * _LANES)
                off = lax.rem(pos, _F) * _V
                sl = (j, pl.ds(v * _LANES, _LANES))
                idx_v[sl] = idx_v[sl] + off
        copies = [
            pltpu.async_copy(
                tab_hbm.at[idx_v.at[j]],
                rows_v.at[pl.ds(j * _BLK, _BLK)],
                sem,
            )
            for j in range(_CHUNK_BLKS)
        ]
        for cp in copies:
            cp.wait()
        pltpu.sync_copy(rows_v, out_hbm.at[pl.ds(gb * _BLK, _CROWS)])
        return carry

    lax.fori_loop(0, _NCHUNK, chunk, 0)


@functools.partial(jax.jit, static_argnums=())
def kernel(x, tables):
    x2 = jnp.asarray(x, jnp.int32).reshape(_N // _BLK, _BLK)
    tab = tables.reshape(_F * _V, _D)
    mesh = plsc.VectorSubcoreMesh(core_axis_name="c", subcore_axis_name="s")
    out = pl.kernel(
        _gather_body,
        mesh=mesh,
        out_type=jax.ShapeDtypeStruct((_N, _D), jnp.float32),
        scratch_types=[
            pltpu.VMEM((_CHUNK_BLKS, _BLK), jnp.int32),
            pltpu.VMEM((_CROWS, _D), jnp.float32),
            pltpu.SemaphoreType.DMA,
        ],
        compiler_params=pltpu.CompilerParams(use_tc_tiling_on_sc=False),
    )(x2, tab)
    return out.reshape(_B, _F * _D)


# speedup vs baseline: 4.3215x; 3.5995x over previous
import functools

import jax
import jax.numpy as jnp
from jax import lax
from jax.experimental import pallas as pl
from jax.experimental.pallas import tpu as pltpu
from jax.experimental.pallas import tpu_sc as plsc

_B = 16384
_F = 26
_V = 100000
_D = 32
_R = _F * _D
_NC = 2
_NS = 16
_NW = _NC * _NS
_TASKS_PW = _R // _NW
_LANES = 16


def _gather_body(xp_hbm, tab_hbm, out_hbm, col_v, idx_v, row_v, sem):
    wid = lax.axis_index("s") * _NC + lax.axis_index("c")

    def task(t, carry):
        f = t
        d = wid
        r = f * _D + d
        pltpu.sync_copy(tab_hbm.at[f, d], col_v)
        for h in range(2):
            pltpu.sync_copy(xp_hbm.at[f, pl.ds(64 * h, 64)], idx_v)

            def vec(rr, carry2, _h=h):
                for cc in range(8):
                    csl = pl.ds(cc * _LANES, _LANES)
                    idx16 = idx_v[rr, csl]
                    row_v[64 * _h + rr, csl] = plsc.load_gather(col_v, [idx16])
                return carry2

            lax.fori_loop(0, 64, vec, 0)
        pltpu.sync_copy(row_v, out_hbm.at[r])
        return carry

    lax.fori_loop(0, _TASKS_PW, task, 0)


def kernel(x, tables):
    xp = jnp.swapaxes(jnp.asarray(x, jnp.int32), 0, 1).reshape(_F, 128, 128)
    tab = jnp.swapaxes(tables, 1, 2)
    mesh = plsc.VectorSubcoreMesh(core_axis_name="c", subcore_axis_name="s")
    out_t = pl.kernel(
        _gather_body,
        mesh=mesh,
        out_type=jax.ShapeDtypeStruct((_R, 128, 128), jnp.float32),
        scratch_types=[
            pltpu.VMEM((_V,), jnp.float32),
            pltpu.VMEM((64, 128), jnp.int32),
            pltpu.VMEM((128, 128), jnp.float32),
            pltpu.SemaphoreType.DMA,
        ],
        compiler_params=pltpu.CompilerParams(
            use_tc_tiling_on_sc=True, needs_layout_passes=False
        ),
    )(xp, tab)
    return out_t.reshape(_R, _B).T


# speedup vs baseline: 5.0225x; 1.1622x over previous
import functools

import jax
import jax.numpy as jnp
from jax import lax
from jax.experimental import pallas as pl
from jax.experimental.pallas import tpu as pltpu
from jax.experimental.pallas import tpu_sc as plsc

_B = 16384
_F = 26
_V = 100000
_D = 32
_R = _F * _D
_NC = 2
_NS = 16
_NW = _NC * _NS
_TASKS_PW = _R // _NW
_LANES = 16


def _gather_body(xp_hbm, tab_hbm, out_hbm, col_v, idx_v, row_v, wsem, _sem):
    wid = lax.axis_index("s") * _NC + lax.axis_index("c")
    d = wid
    trw = lax.div(wid, 8)
    s = lax.rem(wid, 8)

    def out_dst(t):
        return out_hbm.at[4 * t + trw, :, s, :]

    def task(t, carry):
        f = t
        pltpu.sync_copy(tab_hbm.at[f, d], col_v)
        @pl.when(t > 0)
        def _():
            pltpu.make_async_copy(row_v, out_dst(t - 1), wsem).wait()

        for h in range(2):
            pltpu.sync_copy(xp_hbm.at[f, pl.ds(64 * h, 64)], idx_v)

            def vec(rr, carry2, _h=h):
                for cc in range(8):
                    csl = pl.ds(cc * _LANES, _LANES)
                    idx16 = idx_v[rr, csl]
                    row_v[64 * _h + rr, csl] = plsc.load_gather(col_v, [idx16])
                return carry2

            lax.fori_loop(0, 64, vec, 0)
        pltpu.async_copy(row_v, out_dst(t), wsem)
        return carry

    lax.fori_loop(0, _TASKS_PW, task, 0)
    pltpu.make_async_copy(row_v, out_dst(_TASKS_PW - 1), wsem).wait()


def kernel(x, tables):
    xp = jnp.swapaxes(jnp.asarray(x, jnp.int32), 0, 1).reshape(_F, 128, 128)
    tab = jnp.swapaxes(tables, 1, 2)
    mesh = plsc.VectorSubcoreMesh(core_axis_name="c", subcore_axis_name="s")
    out4 = pl.kernel(
        _gather_body,
        mesh=mesh,
        out_type=jax.ShapeDtypeStruct((_R // 8, 128, 8, 128), jnp.float32),
        scratch_types=[
            pltpu.VMEM((_V,), jnp.float32),
            pltpu.VMEM((64, 128), jnp.int32),
            pltpu.VMEM((128, 128), jnp.float32),
            pltpu.SemaphoreType.DMA,
            pltpu.SemaphoreType.DMA,
        ],
        compiler_params=pltpu.CompilerParams(
            use_tc_tiling_on_sc=True, needs_layout_passes=False
        ),
    )(xp, tab)
    return out4.transpose(1, 3, 0, 2).reshape(_B, _R)


# speedup vs baseline: 6.5309x; 1.3003x over previous
import functools

import jax
import jax.numpy as jnp
from jax import lax
from jax.experimental import pallas as pl
from jax.experimental.pallas import tpu as pltpu
from jax.experimental.pallas import tpu_sc as plsc

_B = 16384
_F = 26
_V = 100000
_D = 32
_R = _F * _D
_NC = 2
_NS = 16
_NW = _NC * _NS
_TASKS_PW = _R // _NW
_LANES = 16


def _gather_body(xp_hbm, tab_hbm, out_hbm, col_v, idx_v, row_v, wsem, _sem):
    wid = lax.axis_index("s") * _NC + lax.axis_index("c")
    d = wid
    trw = lax.div(wid, 8)
    s = lax.rem(wid, 8)

    def out_dst(t):
        return out_hbm.at[4 * t + trw, :, s, :]

    def task(t, carry):
        f = t
        pltpu.sync_copy(tab_hbm.at[f, d], col_v)
        @pl.when(t > 0)
        def _():
            pltpu.make_async_copy(row_v, out_dst(t - 1), wsem).wait()

        for h in range(2):
            pltpu.sync_copy(xp_hbm.at[f, pl.ds(64 * h, 64)], idx_v)

            @plsc.parallel_loop(0, 64, unroll=4)
            def _vec(rr, _h=h):
                for cc in range(8):
                    csl = pl.ds(cc * _LANES, _LANES)
                    idx16 = idx_v[rr, csl]
                    row_v[64 * _h + rr, csl] = plsc.load_gather(col_v, [idx16])
        pltpu.async_copy(row_v, out_dst(t), wsem)
        return carry

    lax.fori_loop(0, _TASKS_PW, task, 0)
    pltpu.make_async_copy(row_v, out_dst(_TASKS_PW - 1), wsem).wait()


def kernel(x, tables):
    xp = jnp.swapaxes(jnp.asarray(x, jnp.int32), 0, 1).reshape(_F, 128, 128)
    tab = jnp.swapaxes(tables, 1, 2)
    mesh = plsc.VectorSubcoreMesh(core_axis_name="c", subcore_axis_name="s")
    out4 = pl.kernel(
        _gather_body,
        mesh=mesh,
        out_type=jax.ShapeDtypeStruct((_R // 8, 128, 8, 128), jnp.float32),
        scratch_types=[
            pltpu.VMEM((_V,), jnp.float32),
            pltpu.VMEM((64, 128), jnp.int32),
            pltpu.VMEM((128, 128), jnp.float32),
            pltpu.SemaphoreType.DMA,
            pltpu.SemaphoreType.DMA,
        ],
        compiler_params=pltpu.CompilerParams(
            use_tc_tiling_on_sc=True, needs_layout_passes=False
        ),
    )(xp, tab)
    return out4.transpose(1, 3, 0, 2).reshape(_B, _R)


# speedup vs baseline: 6.7596x; 1.0350x over previous
import functools

import jax
import jax.numpy as jnp
from jax import lax
from jax.experimental import pallas as pl
from jax.experimental.pallas import tpu as pltpu
from jax.experimental.pallas import tpu_sc as plsc

_B = 16384
_F = 26
_V = 100000
_D = 32
_R = _F * _D
_NC = 2
_NS = 16
_NW = _NC * _NS
_TASKS_PW = _R // _NW
_LANES = 16


def _gather_body(xp_hbm, tab_hbm, out_hbm, col_v, idx_v, row_v, wsem, isem):
    wid = lax.axis_index("s") * _NC + lax.axis_index("c")
    d = wid
    trw = lax.div(wid, 8)
    s = lax.rem(wid, 8)

    def out_dst(t):
        return out_hbm.at[4 * t + trw, :, s, :]

    def idx_src(f, k):
        return xp_hbm.at[f, pl.ds(32 * k, 32)]

    pltpu.async_copy(idx_src(0, 0), idx_v.at[0], isem)

    def task(t, carry):
        f = t
        pltpu.sync_copy(tab_hbm.at[f, d], col_v)
        @pl.when(t > 0)
        def _():
            pltpu.make_async_copy(row_v, out_dst(t - 1), wsem).wait()

        for k in range(4):
            buf = k % 2
            pltpu.make_async_copy(idx_src(f, k), idx_v.at[buf], isem).wait()
            if k < 3:
                pltpu.async_copy(idx_src(f, k + 1), idx_v.at[1 - buf], isem)
            else:
                fn = lax.min(f + 1, _TASKS_PW - 1)
                pltpu.async_copy(idx_src(fn, 0), idx_v.at[1 - buf], isem)

            @plsc.parallel_loop(0, 32, unroll=4)
            def _vec(rr, _k=k, _buf=buf):
                for cc in range(8):
                    csl = pl.ds(cc * _LANES, _LANES)
                    idx16 = idx_v[_buf, rr, csl]
                    row_v[32 * _k + rr, csl] = plsc.load_gather(col_v, [idx16])

        pltpu.async_copy(row_v, out_dst(t), wsem)
        return carry

    lax.fori_loop(0, _TASKS_PW, task, 0)
    pltpu.make_async_copy(row_v, out_dst(_TASKS_PW - 1), wsem).wait()
    pltpu.make_async_copy(idx_src(0, 0), idx_v.at[0], isem).wait()


def kernel(x, tables):
    xp = jnp.swapaxes(jnp.asarray(x, jnp.int32), 0, 1).reshape(_F, 128, 128)
    tab = jnp.swapaxes(tables, 1, 2)
    mesh = plsc.VectorSubcoreMesh(core_axis_name="c", subcore_axis_name="s")
    out4 = pl.kernel(
        _gather_body,
        mesh=mesh,
        out_type=jax.ShapeDtypeStruct((_R // 8, 128, 8, 128), jnp.float32),
        scratch_types=[
            pltpu.VMEM((_V,), jnp.float32),
            pltpu.VMEM((2, 32, 128), jnp.int32),
            pltpu.VMEM((128, 128), jnp.float32),
            pltpu.SemaphoreType.DMA,
            pltpu.SemaphoreType.DMA,
        ],
        compiler_params=pltpu.CompilerParams(
            use_tc_tiling_on_sc=True, needs_layout_passes=False
        ),
    )(xp, tab)
    return out4.transpose(1, 3, 0, 2).reshape(_B, _R)
